# baseline (speedup 1.0000x reference)
import math

import jax
import jax.numpy as jnp
from jax.experimental import pallas as pl
from jax.experimental.pallas import tpu as pltpu

S = 2048
H = 768
NH = 12
NKV = 4
HD = 64
HALF = HD // 2
I = 768
E = 64
THETA = 150000.0
EPS = 1e-6
BLK = 256
GRID = S // BLK
REP = NH // NKV
Q_SIZE = NH * HD
KV_SIZE = NKV * HD

_NEG = -1e30
_NT = (((1,), (1,)), ((), ()))


def _split_halves(x, nheads):
    h1 = [x[:, h * HD:h * HD + HALF] for h in range(nheads)]
    h2 = [x[:, h * HD + HALF:(h + 1) * HD] for h in range(nheads)]
    return jnp.concatenate(h1 + h2, axis=1)


def _body(pos_ref, hid_ref, res_ref, wqkv_ref, bqkv_ref, ln1_ref,
          wo_ref, bo_ref, ln2_ref, wr_ref, br_ref,
          wgu_ref, bgu_ref, wd_ref, bd_ref,
          out_ref, r2_out,
          qs, ks, vs, o_sc, wqkv_bf, wo_bf, wr_bf, wgu_bf, wd_bf):
    i = pl.program_id(0)

    @pl.when(i == 0)
    def _cast_weights():
        wqkv_bf[...] = wqkv_ref[...].astype(jnp.bfloat16)
        wo_bf[...] = wo_ref[...].astype(jnp.bfloat16)
        wr_bf[...] = wr_ref[...].astype(jnp.bfloat16)
        wgu_bf[...] = wgu_ref[...].astype(jnp.bfloat16)
        wd_bf[...] = wd_ref[...].astype(jnp.bfloat16)

    @pl.when(i < GRID)
    def _phase_a():
        x = hid_ref[...] + res_ref[...]
        rows = pl.ds(i * BLK, BLK)
        ms = jnp.mean(x * x, axis=1, keepdims=True)
        h = x * jax.lax.rsqrt(ms + EPS) * ln1_ref[...]
        qkv = jax.lax.dot_general(
            h.astype(jnp.bfloat16), wqkv_bf[...], _NT,
            preferred_element_type=jnp.float32) + bqkv_ref[...]

        pos = pos_ref[...]
        jq = jax.lax.rem(
            jax.lax.broadcasted_iota(jnp.int32, (1, NH * HALF), 1),
            HALF).astype(jnp.float32)
        inv_freq = jnp.exp(jq * (-math.log(THETA) / HALF))
        f = pos * inv_freq
        cos_q = jnp.cos(f)
        sin_q = jnp.sin(f)
        cos_k = cos_q[:, :NKV * HALF]
        sin_k = sin_q[:, :NKV * HALF]

        qh = _split_halves(qkv[:, :Q_SIZE], NH)
        kh = _split_halves(qkv[:, Q_SIZE:Q_SIZE + KV_SIZE], NKV)
        q1 = qh[:, :NH * HALF]
        q2 = qh[:, NH * HALF:]
        k1 = kh[:, :NKV * HALF]
        k2 = kh[:, NKV * HALF:]

        scale = HD ** -0.5 * math.log2(math.e)
        qs[rows, :] = (jnp.concatenate(
            [q1 * cos_q - q2 * sin_q, q2 * cos_q + q1 * sin_q],
            axis=1) * scale).astype(jnp.bfloat16)
        ks[rows, :] = jnp.concatenate(
            [k1 * cos_k - k2 * sin_k, k2 * cos_k + k1 * sin_k],
            axis=1).astype(jnp.bfloat16)
        vs[rows, :] = qkv[:, Q_SIZE + KV_SIZE:].astype(jnp.bfloat16)

    def _attn(j):
        W = (j + 1) * BLK
        q0 = j * BLK
        R = REP * BLK

        row = jax.lax.rem(
            jax.lax.broadcasted_iota(jnp.int32, (R, 1), 0), BLK)
        col = jax.lax.broadcasted_iota(jnp.int32, (1, BLK), 1)
        tri = col <= row

        q_blk = qs[pl.ds(q0, BLK), :]
        o_cols = []
        for g in range(NKV):
            hs = []
            for hh in range(REP):
                h = g * REP + hh
                hs.append(jnp.concatenate(
                    [q_blk[:, h * HALF:(h + 1) * HALF],
                     q_blk[:, NH * HALF + h * HALF:
                           NH * HALF + (h + 1) * HALF]],
                    axis=1))
            q_g = jnp.concatenate(hs, axis=0)

            k_g = jnp.concatenate(
                [ks[:W, g * HALF:(g + 1) * HALF],
                 ks[:W, NKV * HALF + g * HALF:NKV * HALF + (g + 1) * HALF]],
                axis=1)
            s = jax.lax.dot_general(q_g, k_g, _NT,
                                    preferred_element_type=jnp.float32)
            s_tail = jnp.where(tri, s[:, q0:], _NEG)
            m_tail = jnp.max(s_tail, axis=1, keepdims=True)
            if j > 0:
                s_head = s[:, :q0]
                m = jnp.maximum(
                    jnp.max(s_head, axis=1, keepdims=True), m_tail)
                p_head = jnp.exp2(s_head - m)
                p_tail = jnp.exp2(s_tail - m)
                l = (jnp.sum(p_head, axis=1, keepdims=True)
                     + jnp.sum(p_tail, axis=1, keepdims=True))
                pv = (jnp.dot(p_head.astype(jnp.bfloat16),
                              vs[:q0, g * HD:(g + 1) * HD],
                              preferred_element_type=jnp.float32)
                      + jnp.dot(p_tail.astype(jnp.bfloat16),
                                vs[q0:W, g * HD:(g + 1) * HD],
                                preferred_element_type=jnp.float32))
            else:
                m = m_tail
                p_tail = jnp.exp2(s_tail - m)
                l = jnp.sum(p_tail, axis=1, keepdims=True)
                pv = jnp.dot(p_tail.astype(jnp.bfloat16),
                             vs[:BLK, g * HD:(g + 1) * HD],
                             preferred_element_type=jnp.float32)
            o_g = pv / l
            for hh in range(REP):
                o_cols.append(o_g[hh * BLK:(hh + 1) * BLK, :])
        o_sc[...] = jnp.concatenate(o_cols, axis=1).astype(jnp.bfloat16)

    for jj in range(GRID):
        @pl.when(i == GRID + jj)
        def _attn_j(jj=jj):
            _attn(jj)

    @pl.when(i >= GRID)
    def _phase_b():
        attn = jax.lax.dot_general(
            o_sc[...], wo_bf[...], _NT,
            preferred_element_type=jnp.float32) + bo_ref[...]
        r2 = attn + (hid_ref[...] + res_ref[...])
        r2_out[...] = r2

        ms = jnp.mean(r2 * r2, axis=1, keepdims=True)
        h2 = (r2 * jax.lax.rsqrt(ms + EPS) * ln2_ref[...]).astype(
            jnp.bfloat16)

        logits = jax.lax.dot_general(
            h2, wr_bf[...], _NT,
            preferred_element_type=jnp.float32) + br_ref[...]
        m1 = jnp.max(logits, axis=1, keepdims=True)
        s2 = jnp.max(jnp.where(logits >= m1, _NEG, logits),
                     axis=1, keepdims=True)
        e2 = jnp.exp(s2 - m1)
        denom = 1.0 + e2
        factor = 1.0 / denom + e2 / denom

        gu = jax.lax.dot_general(
            h2, wgu_bf[...], _NT,
            preferred_element_type=jnp.float32) + bgu_ref[...]
        gate = gu[:, :I]
        up = gu[:, I:]
        x = gate * (up * jax.nn.sigmoid(up))
        eo = jax.lax.dot_general(
            x.astype(jnp.bfloat16), wd_bf[...], _NT,
            preferred_element_type=jnp.float32) + bd_ref[...]
        out_ref[...] = factor * eo


def kernel(positions, hidden_states, residual, w_qkv, b_qkv, w_o, b_o,
           w_router, b_router, w_gate_up, b_gate_up, w_down, b_down,
           ln1_w, ln2_w):
    f32 = jnp.float32
    bf16 = jnp.bfloat16
    pos = positions.astype(f32).reshape(S, 1)

    full = lambda shape: pl.BlockSpec(shape, lambda i: (0, 0))
    blk_a = lambda cols: pl.BlockSpec(
        (BLK, cols), lambda i: (jnp.where(i < GRID, i, i - GRID), 0))
    blk_b = lambda cols: pl.BlockSpec(
        (BLK, cols), lambda i: (jnp.maximum(i - GRID, 0), 0))

    out, r2 = pl.pallas_call(
        _body,
        grid=(2 * GRID,),
        in_specs=[
            blk_a(1),
            blk_a(H),
            blk_a(H),
            full((Q_SIZE + 2 * KV_SIZE, H)),
            full((1, Q_SIZE + 2 * KV_SIZE)),
            full((1, H)),
            full((H, Q_SIZE)),
            full((1, H)),
            full((1, H)),
            full((E, H)),
            full((1, E)),
            full((2 * I, H)),
            full((1, 2 * I)),
            full((H, I)),
            full((1, H)),
        ],
        out_specs=[blk_b(H), blk_b(H)],
        out_shape=[
            jax.ShapeDtypeStruct((S, H), f32),
            jax.ShapeDtypeStruct((S, H), f32),
        ],
        scratch_shapes=[
            pltpu.VMEM((S, Q_SIZE), bf16),
            pltpu.VMEM((S, KV_SIZE), bf16),
            pltpu.VMEM((S, KV_SIZE), bf16),
            pltpu.VMEM((BLK, Q_SIZE), bf16),
            pltpu.VMEM((Q_SIZE + 2 * KV_SIZE, H), bf16),
            pltpu.VMEM((H, Q_SIZE), bf16),
            pltpu.VMEM((E, H), bf16),
            pltpu.VMEM((2 * I, H), bf16),
            pltpu.VMEM((H, I), bf16),
        ],
    )(pos, hidden_states, residual, w_qkv,
      b_qkv.reshape(1, -1).astype(f32), ln1_w.reshape(1, H).astype(f32),
      w_o, b_o.reshape(1, H).astype(f32), ln2_w.reshape(1, H).astype(f32),
      w_router, b_router.reshape(1, E).astype(f32),
      w_gate_up, b_gate_up.reshape(1, 2 * I).astype(f32),
      w_down, b_down.reshape(1, H).astype(f32))

    return (out, r2)


# speedup vs baseline: 6.1708x; 6.1708x over previous
import math

import jax
import jax.numpy as jnp
from jax.experimental import pallas as pl
from jax.experimental.pallas import tpu as pltpu

S = 2048
H = 768
NH = 12
NKV = 4
HD = 64
HALF = HD // 2
I = 768
E = 64
THETA = 150000.0
EPS = 1e-6
BLK = 256
GRID = S // BLK
REP = NH // NKV
Q_SIZE = NH * HD
KV_SIZE = NKV * HD

_NEG = -1e30
_NT = (((1,), (1,)), ((), ()))


def _split_halves(x, nheads):
    h1 = [x[:, h * HD:h * HD + HALF] for h in range(nheads)]
    h2 = [x[:, h * HD + HALF:(h + 1) * HD] for h in range(nheads)]
    return jnp.concatenate(h1 + h2, axis=1)


def _body(pos_ref, hid_ref, res_ref, wqkv_ref, bqkv_ref, ln1_ref,
          wo_ref, bo_ref, ln2_ref, wr_ref, br_ref,
          wgu_ref, bgu_ref, wd_ref, bd_ref,
          out_ref, r2_out,
          qs, ks, vs, r1s, o_sc, wqkv_bf, wo_bf, wr_bf, wgu_bf, wd_bf):
    i = pl.program_id(0)

    @pl.when(i == 0)
    def _cast_weights():
        wqkv_bf[...] = wqkv_ref[...].astype(jnp.bfloat16)
        wo_bf[...] = wo_ref[...].astype(jnp.bfloat16)
        wr_bf[...] = wr_ref[...].astype(jnp.bfloat16)
        wgu_bf[...] = wgu_ref[...].astype(jnp.bfloat16)
        wd_bf[...] = wd_ref[...].astype(jnp.bfloat16)

    @pl.when(i < GRID)
    def _phase_a():
        x = hid_ref[...] + res_ref[...]
        rows = pl.ds(i * BLK, BLK)
        r1s[rows, :] = x
        ms = jnp.mean(x * x, axis=1, keepdims=True)
        h = x * jax.lax.rsqrt(ms + EPS) * ln1_ref[...]
        qkv = jax.lax.dot_general(
            h.astype(jnp.bfloat16), wqkv_bf[...], _NT,
            preferred_element_type=jnp.float32) + bqkv_ref[...]

        pos = pos_ref[...]
        jq = jax.lax.rem(
            jax.lax.broadcasted_iota(jnp.int32, (1, NH * HALF), 1),
            HALF).astype(jnp.float32)
        inv_freq = jnp.exp(jq * (-math.log(THETA) / HALF))
        f = pos * inv_freq
        cos_q = jnp.cos(f)
        sin_q = jnp.sin(f)
        cos_k = cos_q[:, :NKV * HALF]
        sin_k = sin_q[:, :NKV * HALF]

        qh = _split_halves(qkv[:, :Q_SIZE], NH)
        kh = _split_halves(qkv[:, Q_SIZE:Q_SIZE + KV_SIZE], NKV)
        q1 = qh[:, :NH * HALF]
        q2 = qh[:, NH * HALF:]
        k1 = kh[:, :NKV * HALF]
        k2 = kh[:, NKV * HALF:]

        scale = HD ** -0.5 * math.log2(math.e)
        qs[rows, :] = (jnp.concatenate(
            [q1 * cos_q - q2 * sin_q, q2 * cos_q + q1 * sin_q],
            axis=1) * scale).astype(jnp.bfloat16)
        ks[rows, :] = jnp.concatenate(
            [k1 * cos_k - k2 * sin_k, k2 * cos_k + k1 * sin_k],
            axis=1).astype(jnp.bfloat16)
        vs[rows, :] = qkv[:, Q_SIZE + KV_SIZE:].astype(jnp.bfloat16)

    def _attn(W):
        j = i - GRID
        q0 = j * BLK
        R = REP * BLK

        row = jax.lax.rem(
            jax.lax.broadcasted_iota(jnp.int32, (R, 1), 0), BLK)
        col = jax.lax.broadcasted_iota(jnp.int32, (1, W), 1)
        mask = col <= (q0 + row)

        q_blk = qs[pl.ds(q0, BLK), :]
        o_cols = []
        for g in range(NKV):
            hs = []
            for hh in range(REP):
                h = g * REP + hh
                hs.append(jnp.concatenate(
                    [q_blk[:, h * HALF:(h + 1) * HALF],
                     q_blk[:, NH * HALF + h * HALF:
                           NH * HALF + (h + 1) * HALF]],
                    axis=1))
            q_g = jnp.concatenate(hs, axis=0)

            k_g = jnp.concatenate(
                [ks[:W, g * HALF:(g + 1) * HALF],
                 ks[:W, NKV * HALF + g * HALF:NKV * HALF + (g + 1) * HALF]],
                axis=1)
            v_g = vs[:W, g * HD:(g + 1) * HD]
            s = jax.lax.dot_general(q_g, k_g, _NT,
                                    preferred_element_type=jnp.float32)
            s = jnp.where(mask, s, _NEG)
            m = jnp.max(s, axis=1, keepdims=True)
            p = jnp.exp2(s - m)
            l = jnp.sum(p, axis=1, keepdims=True)
            o_g = jnp.dot(p.astype(jnp.bfloat16), v_g,
                          preferred_element_type=jnp.float32) / l
            for hh in range(REP):
                o_cols.append(o_g[hh * BLK:(hh + 1) * BLK, :])
        o_sc[...] = jnp.concatenate(o_cols, axis=1).astype(jnp.bfloat16)

    for pair in range(GRID // 2):
        @pl.when(jnp.logical_and(i >= GRID + 2 * pair,
                                 i < GRID + 2 * pair + 2))
        def _attn_pair(pair=pair):
            _attn((pair + 1) * 2 * BLK)

    @pl.when(i >= GRID)
    def _phase_b():
        j = i - GRID
        qrows = pl.ds(j * BLK, BLK)

        attn = jax.lax.dot_general(
            o_sc[...], wo_bf[...], _NT,
            preferred_element_type=jnp.float32) + bo_ref[...]
        r2 = attn + r1s[qrows, :]
        r2_out[...] = r2

        ms = jnp.mean(r2 * r2, axis=1, keepdims=True)
        h2 = (r2 * jax.lax.rsqrt(ms + EPS) * ln2_ref[...]).astype(
            jnp.bfloat16)

        logits = jax.lax.dot_general(
            h2, wr_bf[...], _NT,
            preferred_element_type=jnp.float32) + br_ref[...]
        m1 = jnp.max(logits, axis=1, keepdims=True)
        s2 = jnp.max(jnp.where(logits >= m1, _NEG, logits),
                     axis=1, keepdims=True)
        e2 = jnp.exp(s2 - m1)
        denom = 1.0 + e2
        factor = 1.0 / denom + e2 / denom

        gu = jax.lax.dot_general(
            h2, wgu_bf[...], _NT,
            preferred_element_type=jnp.float32) + bgu_ref[...]
        gate = gu[:, :I]
        up = gu[:, I:]
        x = gate * (up * jax.nn.sigmoid(up))
        eo = jax.lax.dot_general(
            x.astype(jnp.bfloat16), wd_bf[...], _NT,
            preferred_element_type=jnp.float32) + bd_ref[...]
        out_ref[...] = factor * eo


def kernel(positions, hidden_states, residual, w_qkv, b_qkv, w_o, b_o,
           w_router, b_router, w_gate_up, b_gate_up, w_down, b_down,
           ln1_w, ln2_w):
    f32 = jnp.float32
    bf16 = jnp.bfloat16
    pos = positions.astype(f32).reshape(S, 1)

    full = lambda shape: pl.BlockSpec(shape, lambda i: (0, 0))
    blk_a = lambda cols: pl.BlockSpec(
        (BLK, cols), lambda i: (jnp.minimum(i, GRID - 1), 0))
    blk_b = lambda cols: pl.BlockSpec(
        (BLK, cols), lambda i: (jnp.maximum(i - GRID, 0), 0))

    out, r2 = pl.pallas_call(
        _body,
        grid=(2 * GRID,),
        in_specs=[
            blk_a(1),
            blk_a(H),
            blk_a(H),
            full((Q_SIZE + 2 * KV_SIZE, H)),
            full((1, Q_SIZE + 2 * KV_SIZE)),
            full((1, H)),
            full((H, Q_SIZE)),
            full((1, H)),
            full((1, H)),
            full((E, H)),
            full((1, E)),
            full((2 * I, H)),
            full((1, 2 * I)),
            full((H, I)),
            full((1, H)),
        ],
        out_specs=[blk_b(H), blk_b(H)],
        out_shape=[
            jax.ShapeDtypeStruct((S, H), f32),
            jax.ShapeDtypeStruct((S, H), f32),
        ],
        scratch_shapes=[
            pltpu.VMEM((S, Q_SIZE), bf16),
            pltpu.VMEM((S, KV_SIZE), bf16),
            pltpu.VMEM((S, KV_SIZE), bf16),
            pltpu.VMEM((S, H), f32),
            pltpu.VMEM((BLK, Q_SIZE), bf16),
            pltpu.VMEM((Q_SIZE + 2 * KV_SIZE, H), bf16),
            pltpu.VMEM((H, Q_SIZE), bf16),
            pltpu.VMEM((E, H), bf16),
            pltpu.VMEM((2 * I, H), bf16),
            pltpu.VMEM((H, I), bf16),
        ],
    )(pos, hidden_states, residual, w_qkv,
      b_qkv.reshape(1, -1).astype(f32), ln1_w.reshape(1, H).astype(f32),
      w_o, b_o.reshape(1, H).astype(f32), ln2_w.reshape(1, H).astype(f32),
      w_router, b_router.reshape(1, E).astype(f32),
      w_gate_up, b_gate_up.reshape(1, 2 * I).astype(f32),
      w_down, b_down.reshape(1, H).astype(f32))

    return (out, r2)
